# baseline (speedup 1.0000x reference)
import functools

import jax
import jax.numpy as jnp
from jax import lax
from jax.experimental import pallas as pl
from jax.experimental.pallas import tpu as pltpu
from jax.experimental.pallas import tpu_sc as plsc

_HIDDEN = 64
_NC = 2
_NS = 16
_NW = _NC * _NS
_CHUNK = 128


@functools.partial(jax.jit, static_argnames=("n_total",))
def _embed_gather(ids2d, table, n_total):
    n_per_w = n_total // _NW
    n_chunks = n_per_w // _CHUNK
    mesh = plsc.VectorSubcoreMesh(core_axis_name="c", subcore_axis_name="s")

    @functools.partial(
        pl.kernel,
        out_type=jax.ShapeDtypeStruct((n_total, _HIDDEN), jnp.float32),
        mesh=mesh,
        scratch_types=[
            pltpu.VMEM((n_chunks, _CHUNK), jnp.int32),
            pltpu.VMEM((_CHUNK, _HIDDEN), jnp.float32),
            pltpu.SemaphoreType.DMA,
        ],
        compiler_params=pltpu.CompilerParams(use_tc_tiling_on_sc=False),
    )
    def k(ids_hbm, table_hbm, out_hbm, idx_v, rows_v, sem):
        wid = lax.axis_index("s") * _NC + lax.axis_index("c")
        row_base = wid * n_chunks
        pltpu.sync_copy(ids_hbm.at[pl.ds(row_base, n_chunks)], idx_v)

        @pl.loop(0, n_chunks)
        def _(j):
            pltpu.async_copy(table_hbm.at[idx_v.at[j]], rows_v, sem).wait()
            out_base = (row_base + j) * _CHUNK
            pltpu.sync_copy(rows_v, out_hbm.at[pl.ds(out_base, _CHUNK)])

    return k(ids2d, table)


def kernel(input_ids, table):
    batch, seq = input_ids.shape
    n_total = batch * seq
    ids2d = input_ids.reshape(n_total // _CHUNK, _CHUNK)
    out = _embed_gather(ids2d, table, n_total)
    return out.reshape(batch, seq, _HIDDEN)


# speedup vs baseline: 1.1151x; 1.1151x over previous
import functools

import jax
import jax.numpy as jnp
from jax import lax
from jax.experimental import pallas as pl
from jax.experimental.pallas import tpu as pltpu
from jax.experimental.pallas import tpu_sc as plsc

_HIDDEN = 64
_NC = 2
_NS = 16
_NW = _NC * _NS
_CHUNK = 128
_NBUF = 8
_LOOK = 4


@functools.partial(jax.jit, static_argnames=("n_total",))
def _embed_gather(ids2d, table, n_total):
    n_per_w = n_total // _NW
    n_chunks = n_per_w // _CHUNK
    assert n_chunks % _NBUF == 0 and n_chunks >= 2 * _NBUF
    mesh = plsc.VectorSubcoreMesh(core_axis_name="c", subcore_axis_name="s")

    @functools.partial(
        pl.kernel,
        out_type=jax.ShapeDtypeStruct((n_total, _HIDDEN), jnp.float32),
        mesh=mesh,
        scratch_types=[
            pltpu.VMEM((n_chunks, _CHUNK), jnp.int32),
            pltpu.VMEM((_NBUF, _CHUNK, _HIDDEN), jnp.float32),
            pltpu.SemaphoreType.DMA((_NBUF,)),
            pltpu.SemaphoreType.DMA((_NBUF,)),
        ],
        compiler_params=pltpu.CompilerParams(use_tc_tiling_on_sc=False),
    )
    def k(ids_hbm, table_hbm, out_hbm, idx_v, rows_v, gsem, ssem):
        wid = lax.axis_index("s") * _NC + lax.axis_index("c")
        row_base = wid * n_chunks
        pltpu.sync_copy(ids_hbm.at[pl.ds(row_base, n_chunks)], idx_v)

        def gather_desc(j, b):
            return pltpu.make_async_copy(
                table_hbm.at[idx_v.at[j]], rows_v.at[b], gsem.at[b])

        def store_desc(j, b):
            out_base = (row_base + j) * _CHUNK
            return pltpu.make_async_copy(
                rows_v.at[b], out_hbm.at[pl.ds(out_base, _CHUNK)], ssem.at[b])

        for j in range(_LOOK):
            gather_desc(j, j % _NBUF).start()

        def slot(j, b, first, last):
            gather_desc(j, b).wait()
            store_desc(j, b).start()
            jn = j + _LOOK
            bn = (b + _LOOK) % _NBUF
            if not first:
                store_desc(jn - _NBUF, bn).wait()
            if not last:
                gather_desc(jn, bn).start()

        for b in range(_NBUF):
            slot(b, b, first=(b < _LOOK), last=False)

        @pl.loop(_NBUF, n_chunks - _NBUF, step=_NBUF)
        def _(g):
            for b in range(_NBUF):
                slot(g + b, b, first=False, last=False)

        g_last = n_chunks - _NBUF
        for b in range(_NBUF):
            slot(g_last + b, b, first=False, last=(b >= _NBUF - _LOOK))

        for b in range(_NBUF - _LOOK, _NBUF):
            store_desc(g_last + b, b).wait()

    return k(ids2d, table)


def kernel(input_ids, table):
    batch, seq = input_ids.shape
    n_total = batch * seq
    ids2d = input_ids.reshape(n_total // _CHUNK, _CHUNK)
    out = _embed_gather(ids2d, table, n_total)
    return out.reshape(batch, seq, _HIDDEN)
